# baseline (speedup 1.0000x reference)
import functools

import jax
import jax.numpy as jnp
from jax import lax
from jax.experimental import pallas as pl
from jax.experimental.pallas import tpu as pltpu
from jax.experimental.pallas import tpu_sc as plsc

_LAG = 24
_QN = 64
_B = 32
_T = 8192
_TOUT = _T - _LAG + 1
_PADT = 8320
_NI = _T // 16 + 1
_TB = 8176

_mesh = plsc.VectorSubcoreMesh(core_axis_name="c", subcore_axis_name="s")


@functools.partial(
    pl.kernel,
    out_type=jax.ShapeDtypeStruct((_B, 1, _T), jnp.int32),
    mesh=_mesh,
    scratch_types=[
        pltpu.VMEM((_T * 4,), jnp.float32),
        pltpu.VMEM((_PADT,), jnp.float32),
        pltpu.VMEM((_PADT,), jnp.float32),
        pltpu.VMEM((_PADT,), jnp.float32),
        pltpu.VMEM((_T,), jnp.int32),
    ],
    compiler_params=pltpu.CompilerParams(needs_layout_passes=False),
)
def _sc_bins(x_hbm, out_hbm, xr, xv, wa, wb, bv):
    wid = lax.axis_index("s") * 2 + lax.axis_index("c")
    iota = lax.iota(jnp.int32, 16)
    ninf = jnp.full((16,), -jnp.inf, dtype=jnp.float32)

    pltpu.sync_copy(x_hbm.at[wid], xr)

    def gbody(i, carry):
        b = i * 64
        for j in range(4):
            xv[pl.ds(b + j * 16, 16)] = plsc.load_gather(
                xr, [(b + j * 16 + iota) * 4 + 1])
        return carry
    lax.fori_loop(0, _T // 64, gbody, 0)

    for b in range(_T, _PADT, 16):
        xv[pl.ds(b, 16)] = ninf
        wa[pl.ds(b, 16)] = ninf
        wb[pl.ds(b, 16)] = ninf

    def mpass(dst, src, off):
        def body(i, carry):
            b = i * 32
            dst[pl.ds(b, 16)] = jnp.maximum(src[pl.ds(b, 16)],
                                            src[pl.ds(b + off, 16)])
            dst[pl.ds(b + 16, 16)] = jnp.maximum(src[pl.ds(b + 16, 16)],
                                                 src[pl.ds(b + 16 + off, 16)])
            return carry
        lax.fori_loop(0, _NI // 2 + 1, body, 0)

    mpass(wa, xv, 1)
    mpass(wb, wa, 2)
    mpass(wa, wb, 4)
    mpass(wb, wa, 8)

    def fbody(i, carry):
        for j in range(2):
            b = i * 32 + j * 16
            v = jnp.maximum(wb[pl.ds(b, 16)], wa[pl.ds(b + 16, 16)])
            u = jnp.clip((v + 2.0) * 16.0, -1.0, 64.0)
            c0 = (u + 1.0).astype(jnp.int32) - 1
            blo = c0.astype(jnp.float32) * 0.0625 - 2.0
            bhi = (c0 + 1).astype(jnp.float32) * 0.0625 - 2.0
            c = c0 - (v < blo).astype(jnp.int32) + (v >= bhi).astype(jnp.int32)
            bv[pl.ds(b, 16)] = jnp.clip(c, -1, 63)
        return carry
    lax.fori_loop(0, _T // 32, fbody, 0)

    pltpu.sync_copy(bv, out_hbm.at[wid, 0])


def _tc_expand_body(bins_ref, out_ref):
    row = bins_ref[0, 0, :]
    c = lax.slice(row, (0,), (_TB,)).reshape(_TB, 1)
    col = lax.broadcasted_iota(jnp.int32, (1, _QN), 1)
    out_ref[0] = (c == col).astype(jnp.float32)


def _tc_expand(bins):
    return pl.pallas_call(
        _tc_expand_body,
        out_shape=jax.ShapeDtypeStruct((_B, _TOUT, _QN), jnp.float32),
        grid=(_B, -(-_TOUT // _TB)),
        in_specs=[pl.BlockSpec((1, 1, _T), lambda b, t: (b, 0, 0))],
        out_specs=pl.BlockSpec((1, _TB, _QN), lambda b, t: (b, t, 0)),
    )(bins)


def kernel(inp):
    bins = _sc_bins(inp.reshape(_B, _T * 4))
    return _tc_expand(bins)


# speedup vs baseline: 1.0125x; 1.0125x over previous
import functools

import jax
import jax.numpy as jnp
from jax import lax
from jax.experimental import pallas as pl
from jax.experimental.pallas import tpu as pltpu
from jax.experimental.pallas import tpu_sc as plsc

_LAG = 24
_QN = 64
_B = 32
_T = 8192
_TOUT = _T - _LAG + 1
_PADT = 8320
_NI = _T // 16 + 1
_TB = 8176

_mesh = plsc.VectorSubcoreMesh(core_axis_name="c", subcore_axis_name="s")

_HT = _T // 2
_HL = 4224
_HNI = (_HT + 32) // 16 + 1


def _make_sc_bins(h):
    @functools.partial(
        pl.kernel,
        out_type=jax.ShapeDtypeStruct((_B // 2, 1, _T), jnp.int32),
        mesh=_mesh,
        scratch_types=[
            pltpu.VMEM((_HT * 4 + 128,), jnp.float32),
            pltpu.VMEM((_HL,), jnp.float32),
            pltpu.VMEM((_HL,), jnp.float32),
            pltpu.VMEM((_HL,), jnp.float32),
            pltpu.VMEM((_HT,), jnp.int32),
        ],
        compiler_params=pltpu.CompilerParams(needs_layout_passes=False),
    )
    def _sc_bins_half(x_hbm, out_hbm, xr, xv, wa, wb, bv):
        wid = lax.axis_index("s") * 2 + lax.axis_index("c")
        b_loc = wid % 16
        h2 = wid // 16
        batch = h * 16 + b_loc
        t0w = h2 * _HT * 4
        iota = lax.iota(jnp.int32, 16)
        ninf = jnp.full((16,), -jnp.inf, dtype=jnp.float32)

        @pl.when(h2 == 0)
        def _():
            pltpu.sync_copy(x_hbm.at[batch, pl.ds(0, _HT * 4 + 128)],
                            xr.at[pl.ds(0, _HT * 4 + 128)])

        @pl.when(h2 == 1)
        def _():
            pltpu.sync_copy(x_hbm.at[batch, pl.ds(_HT * 4, _HT * 4)],
                            xr.at[pl.ds(0, _HT * 4)])

        def gbody(i, carry):
            b = i * 64
            for j in range(4):
                xv[pl.ds(b + j * 16, 16)] = plsc.load_gather(
                    xr, [(b + j * 16 + iota) * 4 + 1])
            return carry
        lax.fori_loop(0, _HT // 64, gbody, 0)
        for b in range(_HT, _HT + 32, 16):
            xv[pl.ds(b, 16)] = plsc.load_gather(xr, [(b + iota) * 4 + 1])

        for b in range(_HT + 32, _HL, 16):
            xv[pl.ds(b, 16)] = ninf
            wa[pl.ds(b, 16)] = ninf
            wb[pl.ds(b, 16)] = ninf

        def mpass(dst, src, off):
            def body(i, carry):
                b = i * 32
                dst[pl.ds(b, 16)] = jnp.maximum(src[pl.ds(b, 16)],
                                                src[pl.ds(b + off, 16)])
                dst[pl.ds(b + 16, 16)] = jnp.maximum(
                    src[pl.ds(b + 16, 16)], src[pl.ds(b + 16 + off, 16)])
                return carry
            lax.fori_loop(0, _HNI // 2 + 1, body, 0)

        mpass(wa, xv, 1)
        mpass(wb, wa, 2)
        mpass(wa, wb, 4)
        mpass(wb, wa, 8)

        def fbody(i, carry):
            for j in range(2):
                b = i * 32 + j * 16
                v = jnp.maximum(wb[pl.ds(b, 16)], wa[pl.ds(b + 16, 16)])
                u = jnp.clip((v + 2.0) * 16.0, -1.0, 64.0)
                c0 = (u + 1.0).astype(jnp.int32) - 1
                blo = c0.astype(jnp.float32) * 0.0625 - 2.0
                bhi = (c0 + 1).astype(jnp.float32) * 0.0625 - 2.0
                c = (c0 - (v < blo).astype(jnp.int32)
                     + (v >= bhi).astype(jnp.int32))
                bv[pl.ds(b, 16)] = jnp.clip(c, -1, 63)
            return carry
        lax.fori_loop(0, _HT // 32, fbody, 0)

        pltpu.sync_copy(bv, out_hbm.at[b_loc, 0, pl.ds(h2 * _HT, _HT)])

    return _sc_bins_half


_sc_bins_0 = _make_sc_bins(0)
_sc_bins_1 = _make_sc_bins(1)


def _tc_expand_body(bins_ref, out_ref):
    row = bins_ref[0, 0, :]
    c = lax.slice(row, (0,), (_TB,)).reshape(_TB, 1)
    col = lax.broadcasted_iota(jnp.int32, (1, _QN), 1)
    out_ref[0] = (c == col).astype(jnp.float32)


def _tc_expand_body_aliased(bins_ref, prev_ref, out_ref):
    del prev_ref
    _tc_expand_body(bins_ref, out_ref)


def _tc_expand_half(bins_h, h, prev=None):
    in_specs = [pl.BlockSpec((1, 1, _T), lambda b: (b, 0, 0))]
    inputs = [bins_h]
    kwargs = {}
    body = _tc_expand_body
    if prev is not None:
        in_specs.append(pl.BlockSpec(memory_space=pltpu.MemorySpace.HBM))
        inputs.append(prev)
        kwargs["input_output_aliases"] = {1: 0}
        body = _tc_expand_body_aliased
    return pl.pallas_call(
        body,
        out_shape=jax.ShapeDtypeStruct((_B, _TOUT, _QN), jnp.float32),
        grid=(_B // 2,),
        in_specs=in_specs,
        out_specs=pl.BlockSpec((1, _TB, _QN), lambda b: (h * 16 + b, 0, 0)),
        **kwargs,
    )(*inputs)


def kernel(inp):
    x = inp.reshape(_B, _T * 4)
    bins0 = _sc_bins_0(x)
    bins1 = _sc_bins_1(x)
    out = _tc_expand_half(bins0, 0)
    return _tc_expand_half(bins1, 1, prev=out)
